# baseline (speedup 1.0000x reference)
import jax
import jax.numpy as jnp
from jax.experimental import pallas as pl
from jax.experimental.pallas import tpu as pltpu

_C = 2048
_NB = 4


def _pe_kernel(x_hbm, pos_hbm, out_hbm, pos_vmem, xbuf,
               load_sem, store_sem, pos_sem):
    R = x_hbm.shape[0]
    Lp = pos_hbm.shape[0]
    n = R // _C

    pltpu.make_async_copy(pos_hbm, pos_vmem, pos_sem).start()

    def load(i):
        slot = i % _NB
        pltpu.make_async_copy(
            x_hbm.at[pl.ds(i * _C, _C)], xbuf.at[slot], load_sem.at[slot]
        ).start()

    for j in range(min(_NB, n)):
        load(j)

    pltpu.make_async_copy(pos_hbm, pos_vmem, pos_sem).wait()

    for i in range(n):
        slot = i % _NB
        pltpu.make_async_copy(
            x_hbm.at[pl.ds(i * _C, _C)], xbuf.at[slot], load_sem.at[slot]
        ).wait()
        off = (i * _C) % Lp
        xbuf[slot] = xbuf[slot] + pos_vmem[pl.ds(off, _C)]
        pltpu.make_async_copy(
            xbuf.at[slot], out_hbm.at[pl.ds(i * _C, _C)], store_sem.at[slot]
        ).start()
        if i + _NB < n:
            pltpu.make_async_copy(
                xbuf.at[slot], out_hbm.at[pl.ds(i * _C, _C)], store_sem.at[slot]
            ).wait()
            load(i + _NB)

    for i in range(max(0, n - _NB), n):
        slot = i % _NB
        pltpu.make_async_copy(
            xbuf.at[slot], out_hbm.at[pl.ds(i * _C, _C)], store_sem.at[slot]
        ).wait()


def kernel(X, pos_embedding):
    B, L, D = X.shape
    out = pl.pallas_call(
        _pe_kernel,
        in_specs=[
            pl.BlockSpec(memory_space=pl.ANY),
            pl.BlockSpec(memory_space=pl.ANY),
        ],
        out_specs=pl.BlockSpec(memory_space=pl.ANY),
        out_shape=jax.ShapeDtypeStruct((B * L, D), X.dtype),
        scratch_shapes=[
            pltpu.VMEM((L, D), X.dtype),
            pltpu.VMEM((_NB, _C, D), X.dtype),
            pltpu.SemaphoreType.DMA((_NB,)),
            pltpu.SemaphoreType.DMA((_NB,)),
            pltpu.SemaphoreType.DMA,
        ],
    )(X.reshape(B * L, D), pos_embedding)
    return out.reshape(B, L, D)


# speedup vs baseline: 1.0622x; 1.0622x over previous
import jax
import jax.numpy as jnp
from jax.experimental import pallas as pl
from jax.experimental.pallas import tpu as pltpu

_C = 2048
_NB = 3


def _pe_kernel(x_hbm, pos_hbm, out_hbm, pos_vmem, xbuf, obuf,
               load_sem, store_sem, pos_sem):
    R = x_hbm.shape[0]
    Lp = pos_hbm.shape[0]
    n = R // _C

    pltpu.make_async_copy(pos_hbm, pos_vmem, pos_sem).start()

    def load(i):
        slot = i % _NB
        pltpu.make_async_copy(
            x_hbm.at[pl.ds(i * _C, _C)], xbuf.at[slot], load_sem.at[slot]
        ).start()

    for j in range(min(_NB, n)):
        load(j)

    pltpu.make_async_copy(pos_hbm, pos_vmem, pos_sem).wait()

    for i in range(n):
        slot = i % _NB
        pltpu.make_async_copy(
            x_hbm.at[pl.ds(i * _C, _C)], xbuf.at[slot], load_sem.at[slot]
        ).wait()
        if i >= _NB:
            pltpu.make_async_copy(
                obuf.at[slot], out_hbm.at[pl.ds((i - _NB) * _C, _C)],
                store_sem.at[slot],
            ).wait()
        off = (i * _C) % Lp
        obuf[slot] = xbuf[slot] + pos_vmem[pl.ds(off, _C)]
        pltpu.make_async_copy(
            obuf.at[slot], out_hbm.at[pl.ds(i * _C, _C)], store_sem.at[slot]
        ).start()
        if i + _NB < n:
            load(i + _NB)

    for i in range(max(0, n - _NB), n):
        slot = i % _NB
        pltpu.make_async_copy(
            obuf.at[slot], out_hbm.at[pl.ds(i * _C, _C)], store_sem.at[slot]
        ).wait()


def kernel(X, pos_embedding):
    B, L, D = X.shape
    out = pl.pallas_call(
        _pe_kernel,
        in_specs=[
            pl.BlockSpec(memory_space=pl.ANY),
            pl.BlockSpec(memory_space=pl.ANY),
        ],
        out_specs=pl.BlockSpec(memory_space=pl.ANY),
        out_shape=jax.ShapeDtypeStruct((B * L, D), X.dtype),
        scratch_shapes=[
            pltpu.VMEM((L, D), X.dtype),
            pltpu.VMEM((_NB, _C, D), X.dtype),
            pltpu.VMEM((_NB, _C, D), X.dtype),
            pltpu.SemaphoreType.DMA((_NB,)),
            pltpu.SemaphoreType.DMA((_NB,)),
            pltpu.SemaphoreType.DMA,
        ],
    )(X.reshape(B * L, D), pos_embedding)
    return out.reshape(B, L, D)
